# baseline (speedup 1.0000x reference)
import functools

import jax
import jax.numpy as jnp
from jax import lax
from jax.experimental import pallas as pl
from jax.experimental.pallas import tpu as pltpu
from jax.experimental.pallas import tpu_sc as plsc

BATCH = 16384
EMBED_DIM = 64
ROWS = 2000000
ITEM_OFFSET = 1000000

NQ = 4
NSLOT = 3

A_STRIDE = (ROWS // 128) * 8 * 128
B_STRIDE = 8 * 128
C_STRIDE = 128


def _sc_dot(xu, xi, tflat):
    info = plsc.get_sparse_core_info()
    nc, ns = info.num_cores, info.num_subcores
    nw = nc * ns
    assert BATCH % (nw * NQ * 16) == 0
    bpw = BATCH // nw
    qrows = bpw // NQ
    qg = qrows // 16
    idx_per_q = qrows * EMBED_DIM

    def body(xu_hbm, xi_hbm, tflat_hbm, out_hbm, *scratch):
        xu_v, xi_v = scratch[0], scratch[1]
        bufs = []
        for s in range(NSLOT):
            bufs.append((scratch[2 + 2 * s], scratch[3 + 2 * s],
                         scratch[2 + 2 * NSLOT + 2 * s],
                         scratch[3 + 2 * NSLOT + 2 * s],
                         scratch[3 + 4 * NSLOT + s]))
        out_v = scratch[2 + 4 * NSLOT]
        semx = scratch[3 + 4 * NSLOT + NSLOT]

        wid = lax.axis_index("s") * nc + lax.axis_index("c")
        base = wid * bpw

        cpu = pltpu.async_copy(xu_hbm.at[pl.ds(base, bpw)], xu_v, semx)
        cpi = pltpu.async_copy(xi_hbm.at[pl.ds(base, bpw)], xi_v, semx)
        cpu.wait()
        cpi.wait()

        def gen_q(q, idx_u, idx_i):
            h0 = q * qrows

            def gen(g, carry):
                u = xu_v[pl.ds(h0 + g * 16, 16)]
                it = xi_v[pl.ds(h0 + g * 16, 16)] + ITEM_OFFSET
                bu = lax.shift_right_logical(u, 7) * B_STRIDE + (u & 127)
                bi = lax.shift_right_logical(it, 7) * B_STRIDE + (it & 127)
                for a in range(8):
                    for c in range(8):
                        off = a * A_STRIDE + c * C_STRIDE
                        p = g * 1024 + (a * 8 + c) * 16
                        idx_u[pl.ds(p, 16)] = bu + off
                        idx_i[pl.ds(p, 16)] = bi + off
                return carry

            lax.fori_loop(0, qg, gen, 0)

        def fire_q(idx_u, idx_i, dst_u, dst_i, sem):
            return [pltpu.async_copy(tflat_hbm.at[idx_u], dst_u, sem),
                    pltpu.async_copy(tflat_hbm.at[idx_i], dst_i, sem)]

        def dot_q(q, dst_u, dst_i):
            h0 = q * qrows

            def dot(g, carry):
                acc = jnp.zeros((16,), jnp.float32)
                for k in range(EMBED_DIM):
                    p = g * 1024 + k * 16
                    acc = acc + dst_u[pl.ds(p, 16)] * dst_i[pl.ds(p, 16)]
                out_v[pl.ds(h0 + g * 16, 16)] = acc
                return carry

            lax.fori_loop(0, qg, dot, 0)

        inflight = {}
        for q in range(NQ):
            slot = q % NSLOT
            if q >= NSLOT:
                for cp in inflight[q - NSLOT]:
                    cp.wait()
                dot_q(q - NSLOT, bufs[slot][2], bufs[slot][3])
            gen_q(q, bufs[slot][0], bufs[slot][1])
            inflight[q] = fire_q(*bufs[slot])
        for q in range(max(0, NQ - NSLOT), NQ):
            slot = q % NSLOT
            for cp in inflight[q]:
                cp.wait()
            dot_q(q, bufs[slot][2], bufs[slot][3])

        pltpu.sync_copy(out_v, out_hbm.at[pl.ds(base, bpw)])

    mesh = plsc.VectorSubcoreMesh(core_axis_name="c", subcore_axis_name="s")
    scratch_types = (
        [pltpu.VMEM((bpw,), jnp.int32)] * 2
        + [pltpu.VMEM((idx_per_q,), jnp.int32)] * (2 * NSLOT)
        + [pltpu.VMEM((idx_per_q,), jnp.float32)] * (2 * NSLOT)
        + [pltpu.VMEM((bpw,), jnp.float32)]
        + [pltpu.SemaphoreType.DMA] * (NSLOT + 1)
    )
    kern = functools.partial(
        pl.kernel,
        out_type=jax.ShapeDtypeStruct((BATCH,), jnp.float32),
        mesh=mesh,
        compiler_params=pltpu.CompilerParams(needs_layout_passes=False,
                                             use_tc_tiling_on_sc=False),
        scratch_types=scratch_types,
    )(body)
    return kern(xu, xi, tflat)


def kernel(x, table):
    x = x.astype(jnp.int32)
    tflat = (table.T.reshape(8, 8, ROWS // 128, 128)
             .transpose(0, 2, 1, 3).reshape(-1))
    y = _sc_dot(x[:, 0], x[:, 1], tflat)
    return y.reshape(BATCH, 1)


# speedup vs baseline: 1.0561x; 1.0561x over previous
import functools

import jax
import jax.numpy as jnp
from jax import lax
from jax.experimental import pallas as pl
from jax.experimental.pallas import tpu as pltpu
from jax.experimental.pallas import tpu_sc as plsc

BATCH = 16384
EMBED_DIM = 64
ROWS = 2000000
ITEM_OFFSET = 1000000

NQ = 4
NSLOT = 3

A_STRIDE = (ROWS // 128) * 8 * 128
B_STRIDE = 8 * 128
C_STRIDE = 128


def _sc_dot(xu, xi, tflat):
    info = plsc.get_sparse_core_info()
    nc, ns = info.num_cores, info.num_subcores
    nw = nc * ns
    assert BATCH % (nw * NQ * 16) == 0
    bpw = BATCH // nw
    qrows = bpw // NQ
    qg = qrows // 16
    idx_per_q = qrows * EMBED_DIM

    def body(xu_hbm, xi_hbm, tflat_hbm, out_hbm, *scratch):
        xu_v, xi_v = scratch[0], scratch[1]
        bufs = []
        for s in range(NSLOT):
            bufs.append((scratch[2 + 2 * s], scratch[3 + 2 * s],
                         scratch[2 + 2 * NSLOT + 2 * s],
                         scratch[3 + 2 * NSLOT + 2 * s],
                         scratch[3 + 4 * NSLOT + s]))
        out_v = scratch[2 + 4 * NSLOT]
        semx = scratch[3 + 4 * NSLOT + NSLOT]

        wid = lax.axis_index("s") * nc + lax.axis_index("c")
        base = wid * bpw

        cpu = pltpu.async_copy(xu_hbm.at[pl.ds(base, bpw)], xu_v, semx)
        cpi = pltpu.async_copy(xi_hbm.at[pl.ds(base, bpw)], xi_v, semx)
        cpu.wait()
        cpi.wait()

        def gen_q(q, idx_u, idx_i):
            h0 = q * qrows

            def gen(g, carry):
                u = xu_v[pl.ds(h0 + g * 16, 16)]
                it = xi_v[pl.ds(h0 + g * 16, 16)] + ITEM_OFFSET
                bu = lax.shift_right_logical(u, 7) * B_STRIDE + (u & 127)
                bi = lax.shift_right_logical(it, 7) * B_STRIDE + (it & 127)

                def gen_a(a, carry2):
                    for c in range(8):
                        off = a * A_STRIDE + c * C_STRIDE
                        p = g * 1024 + a * 128 + c * 16
                        idx_u[pl.ds(p, 16)] = bu + off
                        idx_i[pl.ds(p, 16)] = bi + off
                    return carry2

                return lax.fori_loop(0, 8, gen_a, carry)

            lax.fori_loop(0, qg, gen, 0)

        def fire_q(idx_u, idx_i, dst_u, dst_i, sem):
            return [pltpu.async_copy(tflat_hbm.at[idx_u], dst_u, sem),
                    pltpu.async_copy(tflat_hbm.at[idx_i], dst_i, sem)]

        def dot_q(q, dst_u, dst_i):
            h0 = q * qrows

            def dot(g, carry):
                def dot_k(k, acc):
                    p = g * 1024 + k * 64
                    return (acc
                            + dst_u[pl.ds(p, 16)] * dst_i[pl.ds(p, 16)]
                            + dst_u[pl.ds(p + 16, 16)] * dst_i[pl.ds(p + 16, 16)]
                            + dst_u[pl.ds(p + 32, 16)] * dst_i[pl.ds(p + 32, 16)]
                            + dst_u[pl.ds(p + 48, 16)] * dst_i[pl.ds(p + 48, 16)])

                acc = lax.fori_loop(0, EMBED_DIM // 4, dot_k,
                                    jnp.zeros((16,), jnp.float32))
                out_v[pl.ds(h0 + g * 16, 16)] = acc
                return carry

            lax.fori_loop(0, qg, dot, 0)

        inflight = {}
        for q in range(NQ):
            slot = q % NSLOT
            if q >= NSLOT:
                for cp in inflight[q - NSLOT]:
                    cp.wait()
                dot_q(q - NSLOT, bufs[slot][2], bufs[slot][3])
            gen_q(q, bufs[slot][0], bufs[slot][1])
            inflight[q] = fire_q(*bufs[slot])
        for q in range(max(0, NQ - NSLOT), NQ):
            slot = q % NSLOT
            for cp in inflight[q]:
                cp.wait()
            dot_q(q, bufs[slot][2], bufs[slot][3])

        pltpu.sync_copy(out_v, out_hbm.at[pl.ds(base, bpw)])

    mesh = plsc.VectorSubcoreMesh(core_axis_name="c", subcore_axis_name="s")
    scratch_types = (
        [pltpu.VMEM((bpw,), jnp.int32)] * 2
        + [pltpu.VMEM((idx_per_q,), jnp.int32)] * (2 * NSLOT)
        + [pltpu.VMEM((idx_per_q,), jnp.float32)] * (2 * NSLOT)
        + [pltpu.VMEM((bpw,), jnp.float32)]
        + [pltpu.SemaphoreType.DMA] * (NSLOT + 1)
    )
    kern = functools.partial(
        pl.kernel,
        out_type=jax.ShapeDtypeStruct((BATCH,), jnp.float32),
        mesh=mesh,
        compiler_params=pltpu.CompilerParams(needs_layout_passes=False,
                                             use_tc_tiling_on_sc=False),
        scratch_types=scratch_types,
    )(body)
    return kern(xu, xi, tflat)


def kernel(x, table):
    x = x.astype(jnp.int32)
    tflat = (table.T.reshape(8, 8, ROWS // 128, 128)
             .transpose(0, 2, 1, 3).reshape(-1))
    y = _sc_dot(x[:, 0], x[:, 1], tflat)
    return y.reshape(BATCH, 1)


# speedup vs baseline: 1.0641x; 1.0076x over previous
import functools

import jax
import jax.numpy as jnp
from jax import lax
from jax.experimental import pallas as pl
from jax.experimental.pallas import tpu as pltpu
from jax.experimental.pallas import tpu_sc as plsc

BATCH = 16384
EMBED_DIM = 64
ROWS = 2000000
ITEM_OFFSET = 1000000

NSLOT = 3

A_STRIDE = (ROWS // 128) * 8 * 128
B_STRIDE = 8 * 128
C_STRIDE = 128


def _sc_dot(xf, tflat):
    info = plsc.get_sparse_core_info()
    nc, ns = info.num_cores, info.num_subcores
    nw = nc * ns
    bpw = BATCH // nw
    sizes = [bpw // 8, bpw // 8] + [bpw // 4] * 3
    assert sum(sizes) == bpw and bpw % 128 == 0
    starts = [sum(sizes[:q]) for q in range(len(sizes))]
    qmax = max(sizes)
    idx_per_q = qmax * EMBED_DIM

    def body(xf_hbm, tflat_hbm, out_hbm, *scratch):
        xu_v, xi_v = scratch[0], scratch[1]
        bufs = []
        for s in range(NSLOT):
            bufs.append((scratch[2 + 2 * s], scratch[3 + 2 * s],
                         scratch[2 + 2 * NSLOT + 2 * s],
                         scratch[3 + 2 * NSLOT + 2 * s],
                         scratch[3 + 4 * NSLOT + s]))
        out_v = scratch[2 + 4 * NSLOT]
        semx = scratch[3 + 4 * NSLOT + NSLOT]

        wid = lax.axis_index("s") * nc + lax.axis_index("c")
        base = wid * bpw

        xcps = []
        for bb in range(bpw // 128):
            fb = (wid * (bpw // 128) + bb) * 256
            xcps.append(pltpu.async_copy(
                xf_hbm.at[pl.ds(fb, 128)],
                xu_v.at[pl.ds(bb * 128, 128)], semx))
            xcps.append(pltpu.async_copy(
                xf_hbm.at[pl.ds(fb + 128, 128)],
                xi_v.at[pl.ds(bb * 128, 128)], semx))
        for cp in xcps:
            cp.wait()

        def gen_q(h0, rows, idx_u, idx_i):

            def gen(g, carry):
                u = xu_v[pl.ds(h0 + g * 16, 16)]
                it = xi_v[pl.ds(h0 + g * 16, 16)] + ITEM_OFFSET
                bu = lax.shift_right_logical(u, 7) * B_STRIDE + (u & 127)
                bi = lax.shift_right_logical(it, 7) * B_STRIDE + (it & 127)

                def gen_a(a, carry2):
                    for c in range(8):
                        off = a * A_STRIDE + c * C_STRIDE
                        p = g * 1024 + a * 128 + c * 16
                        idx_u[pl.ds(p, 16)] = bu + off
                        idx_i[pl.ds(p, 16)] = bi + off
                    return carry2

                return lax.fori_loop(0, 8, gen_a, carry)

            lax.fori_loop(0, rows // 16, gen, 0)

        def fire_q(rows, idx_u, idx_i, dst_u, dst_i, sem):
            n = rows * EMBED_DIM
            return [pltpu.async_copy(tflat_hbm.at[idx_u.at[pl.ds(0, n)]],
                                     dst_u.at[pl.ds(0, n)], sem),
                    pltpu.async_copy(tflat_hbm.at[idx_i.at[pl.ds(0, n)]],
                                     dst_i.at[pl.ds(0, n)], sem)]

        def dot_q(h0, rows, dst_u, dst_i):

            def dot(g, carry):
                def dot_k(k, acc):
                    p = g * 1024 + k * 64
                    return (acc
                            + dst_u[pl.ds(p, 16)] * dst_i[pl.ds(p, 16)]
                            + dst_u[pl.ds(p + 16, 16)] * dst_i[pl.ds(p + 16, 16)]
                            + dst_u[pl.ds(p + 32, 16)] * dst_i[pl.ds(p + 32, 16)]
                            + dst_u[pl.ds(p + 48, 16)] * dst_i[pl.ds(p + 48, 16)])

                acc = lax.fori_loop(0, EMBED_DIM // 4, dot_k,
                                    jnp.zeros((16,), jnp.float32))
                out_v[pl.ds(h0 + g * 16, 16)] = acc
                return carry

            lax.fori_loop(0, rows // 16, dot, 0)

        nq = len(sizes)
        inflight = {}
        for q in range(nq):
            slot = q % NSLOT
            if q >= NSLOT:
                for cp in inflight[q - NSLOT]:
                    cp.wait()
                dot_q(starts[q - NSLOT], sizes[q - NSLOT],
                      bufs[slot][2], bufs[slot][3])
            gen_q(starts[q], sizes[q], bufs[slot][0], bufs[slot][1])
            inflight[q] = fire_q(sizes[q], *bufs[slot])
        for q in range(max(0, nq - NSLOT), nq):
            slot = q % NSLOT
            for cp in inflight[q]:
                cp.wait()
            dot_q(starts[q], sizes[q], bufs[slot][2], bufs[slot][3])

        pltpu.sync_copy(out_v, out_hbm.at[pl.ds(base, bpw)])

    mesh = plsc.VectorSubcoreMesh(core_axis_name="c", subcore_axis_name="s")
    scratch_types = (
        [pltpu.VMEM((bpw,), jnp.int32)] * 2
        + [pltpu.VMEM((idx_per_q,), jnp.int32)] * (2 * NSLOT)
        + [pltpu.VMEM((idx_per_q,), jnp.float32)] * (2 * NSLOT)
        + [pltpu.VMEM((bpw,), jnp.float32)]
        + [pltpu.SemaphoreType.DMA] * (NSLOT + 1)
    )
    kern = functools.partial(
        pl.kernel,
        out_type=jax.ShapeDtypeStruct((BATCH,), jnp.float32),
        mesh=mesh,
        compiler_params=pltpu.CompilerParams(needs_layout_passes=False,
                                             use_tc_tiling_on_sc=False),
        scratch_types=scratch_types,
    )(body)
    return kern(xf, tflat)


def kernel(x, table):
    x = x.astype(jnp.int32)
    tflat = (table.T.reshape(8, 8, ROWS // 128, 128)
             .transpose(0, 2, 1, 3).reshape(-1))
    xf = x.T.reshape(2, BATCH // 128, 128).transpose(1, 0, 2).reshape(-1)
    y = _sc_dot(xf, tflat)
    return y.reshape(BATCH, 1)
